# baseline (speedup 1.0000x reference)
import functools

import jax
import jax.numpy as jnp
from jax import lax
from jax.experimental import pallas as pl
from jax.experimental.pallas import tpu as pltpu
from jax.experimental.pallas import tpu_sc as plsc

N = 10000
D = 128
E = 320000

NUM_WORKERS = 32
EPT = E // NUM_WORKERS
CH = 80
NCH = EPT // CH
NGR = CH // 16


def _proj_body(x_ref, w1a_ref, w1b_ref, b1_ref, a_ref, b_ref):
    x = x_ref[...]
    a_ref[...] = (
        jnp.dot(x, w1a_ref[...], preferred_element_type=jnp.float32) + b1_ref[...]
    )
    b_ref[...] = jnp.dot(x, w1b_ref[...], preferred_element_type=jnp.float32)


def _proj(x, w1a, w1b, b1):
    return pl.pallas_call(
        _proj_body,
        out_shape=(
            jax.ShapeDtypeStruct((N, D), jnp.float32),
            jax.ShapeDtypeStruct((N, D), jnp.float32),
        ),
    )(x, w1a, w1b, b1)


@functools.cache
def _edge_logits_fn():
    mesh = plsc.VectorSubcoreMesh(
        core_axis_name="c", subcore_axis_name="s", num_cores=2, num_subcores=16
    )

    @functools.partial(
        pl.kernel,
        out_type=jax.ShapeDtypeStruct((E,), jnp.float32),
        mesh=mesh,
        compiler_params=pltpu.CompilerParams(needs_layout_passes=False),
        scratch_types=[
            pltpu.VMEM((EPT,), jnp.int32),
            pltpu.VMEM((EPT,), jnp.int32),
            pltpu.VMEM((CH, D + 1), jnp.float32),
            pltpu.VMEM((CH, D + 1), jnp.float32),
            pltpu.VMEM((CH, D + 1), jnp.float32),
            pltpu.VMEM((CH, D + 1), jnp.float32),
            pltpu.VMEM((D,), jnp.float32),
            pltpu.VMEM((NGR * (D // 16) * 16,), jnp.float32),
            pltpu.VMEM((EPT,), jnp.float32),
            pltpu.SemaphoreType.DMA,
            pltpu.SemaphoreType.DMA,
            pltpu.SemaphoreType.DMA,
            pltpu.SemaphoreType.DMA,
        ],
    )
    def _edge_logits(a_hbm, b_hbm, src_hbm, tgt_hbm, w2_hbm, out_hbm,
                     src_all, tgt_all, a0, b0, a1, b1, w2_v, partials_v,
                     logits_v, sa0, sb0, sa1, sb1):
        wid = lax.axis_index("s") * 2 + lax.axis_index("c")
        base = wid * EPT
        pltpu.sync_copy(w2_hbm, w2_v)
        pltpu.sync_copy(src_hbm.at[pl.ds(base, EPT)], src_all)
        pltpu.sync_copy(tgt_hbm.at[pl.ds(base, EPT)], tgt_all)
        iota16 = lax.iota(jnp.int32, 16)
        bufs = ((a0, b0, sa0, sb0), (a1, b1, sa1, sb1))

        def issue(c, buf):
            a_rows, b_rows, sem_a, sem_b = buf
            pltpu.async_copy(
                a_hbm.at[src_all.at[pl.ds(c * CH, CH)]],
                a_rows.at[:, pl.ds(0, D)], sem_a)
            pltpu.async_copy(
                b_hbm.at[tgt_all.at[pl.ds(c * CH, CH)]],
                b_rows.at[:, pl.ds(0, D)], sem_b)

        def wait(c, buf):
            a_rows, b_rows, sem_a, sem_b = buf
            pltpu.make_async_copy(
                a_hbm.at[src_all.at[pl.ds(c * CH, CH)]],
                a_rows.at[:, pl.ds(0, D)], sem_a).wait()
            pltpu.make_async_copy(
                b_hbm.at[tgt_all.at[pl.ds(c * CH, CH)]],
                b_rows.at[:, pl.ds(0, D)], sem_b).wait()

        row_vecs = tuple(g * 16 + iota16 for g in range(NGR))

        def compute(c, buf):
            a_rows, b_rows, _, _ = buf

            def k_body(k, accs):
                w2k = w2_v[pl.ds(k * 16, 16)]
                accs = list(accs)
                for j in range(16):
                    dv = jnp.full((16,), k * 16 + j, jnp.int32)
                    w2kj = w2k[j]
                    for g in range(NGR):
                        a = plsc.load_gather(a_rows, [row_vecs[g], dv])
                        b = plsc.load_gather(b_rows, [row_vecs[g], dv])
                        h = jnp.maximum(a + b, 0.0)
                        accs[g] = accs[g] + h * w2kj
                return tuple(accs)

            accs = lax.fori_loop(
                0, D // 16, k_body,
                tuple(jnp.zeros((16,), jnp.float32) for _ in range(NGR)),
            )
            for g in range(NGR):
                logits_v[pl.ds(c * CH + g * 16, 16)] = accs[g]

        issue(0, bufs[0])

        def pair_body(p, _):
            c0 = 2 * p
            issue(c0 + 1, bufs[1])
            wait(c0, bufs[0])
            compute(c0, bufs[0])
            issue(c0 + 2, bufs[0])
            wait(c0 + 1, bufs[1])
            compute(c0 + 1, bufs[1])
            return 0

        lax.fori_loop(0, (NCH - 1) // 2, pair_body, 0)
        wait(NCH - 1, bufs[0])
        compute(NCH - 1, bufs[0])
        pltpu.sync_copy(logits_v, out_hbm.at[pl.ds(base, EPT)])

    return _edge_logits


def _softmax_body(x_ref, o_ref):
    x = x_ref[...]
    m = jnp.max(x)
    e = jnp.exp(x - m)
    o_ref[...] = e / jnp.sum(e)


def _softmax(x):
    return pl.pallas_call(
        _softmax_body,
        out_shape=jax.ShapeDtypeStruct(x.shape, jnp.float32),
    )(x)


def kernel(node_embeddings, legal_moves, W1, b1, W2, b2):
    del b2
    a_tab, b_tab = _proj(
        node_embeddings, W1[:D], W1[D:], b1.reshape(1, D)
    )
    logits = _edge_logits_fn()(
        a_tab, b_tab, legal_moves[0], legal_moves[1], W2.reshape(D)
    )
    probs = _softmax(logits.reshape(E // D, D)).reshape(E)
    return probs


# speedup vs baseline: 8.4568x; 8.4568x over previous
import functools

import jax
import jax.numpy as jnp
from jax import lax
from jax.experimental import pallas as pl
from jax.experimental.pallas import tpu as pltpu
from jax.experimental.pallas import tpu_sc as plsc

N = 10000
D = 128
E = 320000

NUM_WORKERS = 32
EPT = E // NUM_WORKERS
CH = 80
NCH = EPT // CH
NGR = CH // 16


def _proj_body(x_ref, w1a_ref, w1b_ref, b1_ref, a_ref, b_ref):
    x = x_ref[...]
    a_ref[...] = (
        jnp.dot(x, w1a_ref[...], preferred_element_type=jnp.float32) + b1_ref[...]
    ).astype(jnp.bfloat16)
    b_ref[...] = jnp.dot(
        x, w1b_ref[...], preferred_element_type=jnp.float32
    ).astype(jnp.bfloat16)


def _proj(x, w1a, w1b, b1):
    return pl.pallas_call(
        _proj_body,
        out_shape=(
            jax.ShapeDtypeStruct((N, D), jnp.bfloat16),
            jax.ShapeDtypeStruct((N, D), jnp.bfloat16),
        ),
    )(x, w1a, w1b, b1)


@functools.cache
def _edge_logits_fn():
    mesh = plsc.VectorSubcoreMesh(
        core_axis_name="c", subcore_axis_name="s", num_cores=2, num_subcores=16
    )

    @functools.partial(
        pl.kernel,
        out_type=jax.ShapeDtypeStruct((E,), jnp.float32),
        mesh=mesh,
        compiler_params=pltpu.CompilerParams(needs_layout_passes=False, use_tc_tiling_on_sc=False),
        scratch_types=[
            pltpu.VMEM((EPT,), jnp.int32),
            pltpu.VMEM((EPT,), jnp.int32),
            pltpu.VMEM((CH, D // 2), jnp.int32),
            pltpu.VMEM((CH, D // 2), jnp.int32),
            pltpu.VMEM((CH, D // 2), jnp.int32),
            pltpu.VMEM((CH, D // 2), jnp.int32),
            pltpu.VMEM((D,), jnp.float32),
            pltpu.VMEM((EPT,), jnp.float32),
            pltpu.SemaphoreType.DMA,
            pltpu.SemaphoreType.DMA,
            pltpu.SemaphoreType.DMA,
            pltpu.SemaphoreType.DMA,
        ],
    )
    def _edge_logits(a_hbm, b_hbm, src_hbm, tgt_hbm, w2_hbm, out_hbm,
                     src_all, tgt_all, a0, b0, a1, b1, w2_v,
                     logits_v, sa0, sb0, sa1, sb1):
        wid = lax.axis_index("s") * 2 + lax.axis_index("c")
        base = wid * EPT
        pltpu.sync_copy(w2_hbm, w2_v)
        pltpu.sync_copy(src_hbm.at[pl.ds(base, EPT)], src_all)
        pltpu.sync_copy(tgt_hbm.at[pl.ds(base, EPT)], tgt_all)
        iota16 = lax.iota(jnp.int32, 16)
        w2e = tuple(w2_v[pl.ds(k * 16, 16)] for k in range(D // 32))
        w2o = tuple(w2_v[pl.ds(D // 2 + k * 16, 16)] for k in range(D // 32))
        bufs = ((a0, b0, sa0, sb0), (a1, b1, sa1, sb1))

        def issue(c, buf):
            a_rows, b_rows, sem_a, sem_b = buf
            pltpu.async_copy(
                a_hbm.at[src_all.at[pl.ds(c * CH, CH)]], a_rows, sem_a)
            pltpu.async_copy(
                b_hbm.at[tgt_all.at[pl.ds(c * CH, CH)]], b_rows, sem_b)

        def wait(c, buf):
            a_rows, b_rows, sem_a, sem_b = buf
            pltpu.make_async_copy(
                a_hbm.at[src_all.at[pl.ds(c * CH, CH)]], a_rows, sem_a).wait()
            pltpu.make_async_copy(
                b_hbm.at[tgt_all.at[pl.ds(c * CH, CH)]], b_rows, sem_b).wait()

        row_vecs = tuple(g * 16 + iota16 for g in range(NGR))

        def compute(c, buf):
            a_rows, b_rows, _, _ = buf

            def group_body(g, _):
                collected = jnp.zeros((16,), jnp.float32)
                for e_local in range(16):
                    e = g * 16 + e_local
                    parts = []
                    for k in range(D // 32):
                        a = plsc.bitcast(
                            a_rows[e, pl.ds(k * 16, 16)], jnp.bfloat16)
                        b = plsc.bitcast(
                            b_rows[e, pl.ds(k * 16, 16)], jnp.bfloat16)
                        h = jnp.maximum(a + b, jnp.bfloat16(0.0))
                        he, ho = plsc.unpack(
                            h, format=plsc.PackFormat.INTERLEAVED)
                        parts.append(he * w2e[k] + ho * w2o[k])
                    while len(parts) > 1:
                        parts = [
                            parts[i] + parts[i + 1]
                            for i in range(0, len(parts), 2)
                        ]
                    s_edge = lax.reduce_sum(parts[0], axes=(0,))
                    collected = jnp.where(iota16 == e_local, s_edge, collected)
                logits_v[pl.ds(c * CH + g * 16, 16)] = collected
                return 0

            lax.fori_loop(0, NGR, group_body, 0)

        issue(0, bufs[0])

        def pair_body(p, _):
            c0 = 2 * p
            issue(c0 + 1, bufs[1])
            wait(c0, bufs[0])
            compute(c0, bufs[0])
            issue(c0 + 2, bufs[0])
            wait(c0 + 1, bufs[1])
            compute(c0 + 1, bufs[1])
            return 0

        lax.fori_loop(0, (NCH - 1) // 2, pair_body, 0)
        wait(NCH - 1, bufs[0])
        compute(NCH - 1, bufs[0])
        pltpu.sync_copy(logits_v, out_hbm.at[pl.ds(base, EPT)])

    return _edge_logits


def _softmax_body(x_ref, o_ref):
    x = x_ref[...]
    m = jnp.max(x)
    e = jnp.exp(x - m)
    o_ref[...] = e / jnp.sum(e)


def _softmax(x):
    return pl.pallas_call(
        _softmax_body,
        out_shape=jax.ShapeDtypeStruct(x.shape, jnp.float32),
    )(x)


def kernel(node_embeddings, legal_moves, W1, b1, W2, b2):
    del b2
    a_tab, b_tab = _proj(
        node_embeddings, W1[:D], W1[D:], b1.reshape(1, D)
    )
    a_i32 = lax.bitcast_convert_type(
        a_tab.reshape(N, D // 2, 2), jnp.int32)
    b_i32 = lax.bitcast_convert_type(
        b_tab.reshape(N, D // 2, 2), jnp.int32)
    w2_flat = W2.reshape(D)
    w2_deint = jnp.concatenate([w2_flat[0::2], w2_flat[1::2]])
    logits = _edge_logits_fn()(
        a_i32, b_i32, legal_moves[0], legal_moves[1], w2_deint
    )
    probs = _softmax(logits.reshape(E // D, D)).reshape(E)
    return probs


# speedup vs baseline: 11.4732x; 1.3567x over previous
import functools

import jax
import jax.numpy as jnp
from jax import lax
from jax.experimental import pallas as pl
from jax.experimental.pallas import tpu as pltpu
from jax.experimental.pallas import tpu_sc as plsc

N = 10000
D = 128
E = 320000

NUM_WORKERS = 32
EPT = E // NUM_WORKERS
CH = 80
NCH = EPT // CH
NGR = CH // 16


def _rne_bf16_bits(x):
    i = lax.bitcast_convert_type(x, jnp.int32)
    t = i + 0x7FFF + jnp.bitwise_and(lax.shift_right_logical(i, 16), 1)
    return lax.shift_right_logical(t, 16)


def _pack_halves(x):
    lo = _rne_bf16_bits(x[:, : D // 2])
    hi = _rne_bf16_bits(x[:, D // 2:])
    return jnp.bitwise_or(lo, lax.shift_left(hi, 16))


def _proj_body(x_ref, w1a_ref, w1b_ref, b1_ref, a_ref, b_ref):
    x = x_ref[...].astype(jnp.bfloat16)
    af = (
        jnp.dot(x, w1a_ref[...].astype(jnp.bfloat16),
                preferred_element_type=jnp.float32) + b1_ref[...]
    )
    bf = jnp.dot(x, w1b_ref[...].astype(jnp.bfloat16),
                 preferred_element_type=jnp.float32)
    a_ref[...] = _pack_halves(af)
    b_ref[...] = _pack_halves(bf)


def _proj(x, w1a, w1b, b1):
    return pl.pallas_call(
        _proj_body,
        out_shape=(
            jax.ShapeDtypeStruct((N, D // 2), jnp.int32),
            jax.ShapeDtypeStruct((N, D // 2), jnp.int32),
        ),
    )(x, w1a, w1b, b1)


@functools.cache
def _edge_logits_fn():
    mesh = plsc.VectorSubcoreMesh(
        core_axis_name="c", subcore_axis_name="s", num_cores=2, num_subcores=16
    )

    @functools.partial(
        pl.kernel,
        out_type=jax.ShapeDtypeStruct((E,), jnp.float32),
        mesh=mesh,
        compiler_params=pltpu.CompilerParams(needs_layout_passes=False, use_tc_tiling_on_sc=False),
        scratch_types=[
            pltpu.VMEM((EPT,), jnp.int32),
            pltpu.VMEM((EPT,), jnp.int32),
            pltpu.VMEM((CH, D // 2), jnp.int32),
            pltpu.VMEM((CH, D // 2), jnp.int32),
            pltpu.VMEM((CH, D // 2), jnp.int32),
            pltpu.VMEM((CH, D // 2), jnp.int32),
            pltpu.VMEM((D,), jnp.float32),
            pltpu.VMEM((EPT,), jnp.float32),
            pltpu.SemaphoreType.DMA,
            pltpu.SemaphoreType.DMA,
            pltpu.SemaphoreType.DMA,
            pltpu.SemaphoreType.DMA,
        ],
    )
    def _edge_logits(a_hbm, b_hbm, src_hbm, tgt_hbm, w2_hbm, out_hbm,
                     src_all, tgt_all, a0, b0, a1, b1, w2_v,
                     logits_v, sa0, sb0, sa1, sb1):
        wid = lax.axis_index("s") * 2 + lax.axis_index("c")
        base = wid * EPT
        pltpu.sync_copy(w2_hbm, w2_v)
        pltpu.sync_copy(src_hbm.at[pl.ds(base, EPT)], src_all)
        pltpu.sync_copy(tgt_hbm.at[pl.ds(base, EPT)], tgt_all)
        iota16 = lax.iota(jnp.int32, 16)
        w2e = tuple(w2_v[pl.ds(k * 16, 16)] for k in range(D // 32))
        w2o = tuple(w2_v[pl.ds(D // 2 + k * 16, 16)] for k in range(D // 32))
        bufs = ((a0, b0, sa0, sb0), (a1, b1, sa1, sb1))

        def issue(c, buf):
            a_rows, b_rows, sem_a, sem_b = buf
            pltpu.async_copy(
                a_hbm.at[src_all.at[pl.ds(c * CH, CH)]], a_rows, sem_a)
            pltpu.async_copy(
                b_hbm.at[tgt_all.at[pl.ds(c * CH, CH)]], b_rows, sem_b)

        def wait(c, buf):
            a_rows, b_rows, sem_a, sem_b = buf
            pltpu.make_async_copy(
                a_hbm.at[src_all.at[pl.ds(c * CH, CH)]], a_rows, sem_a).wait()
            pltpu.make_async_copy(
                b_hbm.at[tgt_all.at[pl.ds(c * CH, CH)]], b_rows, sem_b).wait()

        row_vecs = tuple(g * 16 + iota16 for g in range(NGR))

        def compute(c, buf):
            a_rows, b_rows, _, _ = buf

            def group_body(g, _):
                collected = jnp.zeros((16,), jnp.float32)
                for e_local in range(16):
                    e = g * 16 + e_local
                    parts = []
                    for k in range(D // 32):
                        a = plsc.bitcast(
                            a_rows[e, pl.ds(k * 16, 16)], jnp.bfloat16)
                        b = plsc.bitcast(
                            b_rows[e, pl.ds(k * 16, 16)], jnp.bfloat16)
                        h = jnp.maximum(a + b, jnp.bfloat16(0.0))
                        he, ho = plsc.unpack(
                            h, format=plsc.PackFormat.INTERLEAVED)
                        parts.append(he * w2e[k] + ho * w2o[k])
                    while len(parts) > 1:
                        parts = [
                            parts[i] + parts[i + 1]
                            for i in range(0, len(parts), 2)
                        ]
                    s_edge = lax.reduce_sum(parts[0], axes=(0,))
                    collected = jnp.where(iota16 == e_local, s_edge, collected)
                logits_v[pl.ds(c * CH + g * 16, 16)] = collected
                return 0

            lax.fori_loop(0, NGR, group_body, 0)

        issue(0, bufs[0])

        def pair_body(p, _):
            c0 = 2 * p
            issue(c0 + 1, bufs[1])
            wait(c0, bufs[0])
            compute(c0, bufs[0])
            issue(c0 + 2, bufs[0])
            wait(c0 + 1, bufs[1])
            compute(c0 + 1, bufs[1])
            return 0

        lax.fori_loop(0, (NCH - 1) // 2, pair_body, 0)
        wait(NCH - 1, bufs[0])
        compute(NCH - 1, bufs[0])
        pltpu.sync_copy(logits_v, out_hbm.at[pl.ds(base, EPT)])

    return _edge_logits


def _softmax_body(x_ref, o_ref):
    x = x_ref[...]
    m = jnp.max(x)
    e = jnp.exp(x - m)
    o_ref[...] = e / jnp.sum(e)


def _softmax(x):
    return pl.pallas_call(
        _softmax_body,
        out_shape=jax.ShapeDtypeStruct(x.shape, jnp.float32),
    )(x)


def kernel(node_embeddings, legal_moves, W1, b1, W2, b2):
    del b2
    a_tab, b_tab = _proj(
        node_embeddings, W1[:D], W1[D:], b1.reshape(1, D)
    )
    logits = _edge_logits_fn()(
        a_tab, b_tab, legal_moves[0], legal_moves[1], W2.reshape(D)
    )
    probs = _softmax(logits.reshape(E // D, D)).reshape(E)
    return probs
